# baseline (speedup 1.0000x reference)
import jax
import jax.numpy as jnp
from jax.experimental import pallas as pl
from jax.experimental.pallas import tpu as pltpu

_D_MODEL = 768
_D_EMB = 64
_E = 64
_BASIS = 8
_EB = _E * _BASIS
_TM = 2048
_CHUNKS = 4

_bf16 = jnp.bfloat16


def _chunk(x, wh, flat16, wdec, y_ref, r0, rows):
    f32 = jnp.float32

    e = jax.lax.dot_general(x, wh, (((1,), (0,)), ((), ())),
                            preferred_element_type=f32)
    e = e * (1.0 / (jnp.sqrt(jnp.sum(e * e, axis=1, keepdims=True)) + 1e-8))

    coeffs = jax.lax.dot_general(e.astype(_bf16), flat16, (((1,), (1,)), ((), ())),
                                 preferred_element_type=f32)

    sq = coeffs * coeffs
    s4 = sq[:, 0:4 * _E] + sq[:, 4 * _E:8 * _E]
    s2 = s4[:, 0:2 * _E] + s4[:, 2 * _E:4 * _E]
    energy = s2[:, 0:_E] + s2[:, _E:2 * _E]

    m1 = jnp.max(energy, axis=1, keepdims=True)
    is1 = energy == m1
    en2 = jnp.where(is1, -1.0, energy)
    m2 = jnp.max(en2, axis=1, keepdims=True)
    ed = jnp.exp(m2 - m1)
    g1 = 1.0 / (1.0 + ed)
    g2 = ed * g1
    w = jnp.where(is1, g1, jnp.where(en2 == m2, g2, 0.0))

    w8 = jnp.concatenate([w] * _BASIS, axis=1)
    cw = (coeffs * w8).astype(_bf16)
    mix = jax.lax.dot_general(cw, flat16, (((1,), (0,)), ((), ())),
                              preferred_element_type=f32)

    y = jax.lax.dot_general(mix.astype(_bf16), wdec, (((1,), (0,)), ((), ())),
                            preferred_element_type=f32)
    y_ref[pl.ds(r0, rows), :] = y.astype(_bf16)


def _hrn_block(x_ref, wh_ref, basis_ref, wdec_ref, y_ref, flat_ref):
    @pl.when(pl.program_id(0) == 0)
    def _init():
        basis = basis_ref[...].astype(jnp.float32)
        nb = basis * (1.0 / (
            jnp.sqrt(jnp.sum(basis * basis, axis=1, keepdims=True)) + 1e-8))
        flat_ref[...] = nb.astype(_bf16)

    wh = wh_ref[...]
    wdec = wdec_ref[...]
    flat16 = flat_ref[...]

    rows = _TM // _CHUNKS
    for h in range(_CHUNKS):
        _chunk(x_ref[pl.ds(h * rows, rows), :], wh, flat16, wdec,
               y_ref, h * rows, rows)


@jax.jit
def kernel(x, W_hash, basis, W_dec):
    t = x.shape[0]
    x16 = x.astype(_bf16)
    wh16 = W_hash.astype(_bf16)
    wdec16 = W_dec.astype(_bf16)
    basis2 = basis.transpose(1, 0, 2).reshape(_EB, _D_EMB)
    y16 = pl.pallas_call(
        _hrn_block,
        grid=(t // _TM,),
        in_specs=[
            pl.BlockSpec((_TM, _D_MODEL), lambda i: (i, 0)),
            pl.BlockSpec((_D_MODEL, _D_EMB), lambda i: (0, 0)),
            pl.BlockSpec((_EB, _D_EMB), lambda i: (0, 0)),
            pl.BlockSpec((_D_EMB, _D_MODEL), lambda i: (0, 0)),
        ],
        out_specs=pl.BlockSpec((_TM, _D_MODEL), lambda i: (i, 0)),
        out_shape=jax.ShapeDtypeStruct((t, _D_MODEL), _bf16),
        scratch_shapes=[pltpu.VMEM((_EB, _D_EMB), _bf16)],
    )(x16, wh16, basis2, wdec16)
    return y16.astype(jnp.float32)


# speedup vs baseline: 2.0261x; 2.0261x over previous
import jax
import jax.numpy as jnp
from jax.experimental import pallas as pl
from jax.experimental.pallas import tpu as pltpu

_D_MODEL = 768
_D_EMB = 64
_E = 64
_BASIS = 8
_EB = _E * _BASIS
_TM = 2048
_CHUNKS = 4


def _hrn_block(x_ref, wh_ref, basis_ref, wdec_ref, y_ref, flat_ref):
    f32 = jnp.float32

    @pl.when(pl.program_id(0) == 0)
    def _init():
        basis = basis_ref[...]
        flat_ref[...] = basis * (1.0 / (
            jnp.sqrt(jnp.sum(basis * basis, axis=1, keepdims=True)) + 1e-8))

    wh = wh_ref[...]
    wdec = wdec_ref[...]
    flat = flat_ref[...]

    rows = _TM // _CHUNKS
    R = range(_CHUNKS)

    es = [jax.lax.dot_general(x_ref[pl.ds(h * rows, rows), :], wh,
                              (((1,), (0,)), ((), ())),
                              preferred_element_type=f32) for h in R]
    es = [e * (1.0 / (jnp.sqrt(jnp.sum(e * e, axis=1, keepdims=True)) + 1e-8))
          for e in es]

    cs = [jax.lax.dot_general(e, flat, (((1,), (1,)), ((), ())),
                              preferred_element_type=f32) for e in es]

    sqs = [c * c for c in cs]
    s4s = [s[:, 0:4 * _E] + s[:, 4 * _E:8 * _E] for s in sqs]
    s2s = [s[:, 0:2 * _E] + s[:, 2 * _E:4 * _E] for s in s4s]
    ens = [s[:, 0:_E] + s[:, _E:2 * _E] for s in s2s]

    m1s = [jnp.max(en, axis=1, keepdims=True) for en in ens]
    is1s = [en == m1 for en, m1 in zip(ens, m1s)]
    en2s = [jnp.where(i1, -1.0, en) for i1, en in zip(is1s, ens)]
    m2s = [jnp.max(en2, axis=1, keepdims=True) for en2 in en2s]
    ws = []
    for en2, m1, m2, i1 in zip(en2s, m1s, m2s, is1s):
        ed = jnp.exp(m2 - m1)
        g1 = 1.0 / (1.0 + ed)
        g2 = ed * g1
        ws.append(jnp.where(i1, g1, jnp.where(en2 == m2, g2, 0.0)))

    cws = [c * jnp.concatenate([w] * _BASIS, axis=1) for c, w in zip(cs, ws)]
    mixes = [jax.lax.dot_general(cw, flat, (((1,), (0,)), ((), ())),
                                 preferred_element_type=f32) for cw in cws]

    for h in R:
        y_ref[pl.ds(h * rows, rows), :] = jax.lax.dot_general(
            mixes[h], wdec, (((1,), (0,)), ((), ())), preferred_element_type=f32)


@jax.jit
def kernel(x, W_hash, basis, W_dec):
    t = x.shape[0]
    basis2 = basis.transpose(1, 0, 2).reshape(_EB, _D_EMB)
    return pl.pallas_call(
        _hrn_block,
        grid=(t // _TM,),
        in_specs=[
            pl.BlockSpec((_TM, _D_MODEL), lambda i: (i, 0)),
            pl.BlockSpec((_D_MODEL, _D_EMB), lambda i: (0, 0)),
            pl.BlockSpec((_EB, _D_EMB), lambda i: (0, 0)),
            pl.BlockSpec((_D_EMB, _D_MODEL), lambda i: (0, 0)),
        ],
        out_specs=pl.BlockSpec((_TM, _D_MODEL), lambda i: (i, 0)),
        out_shape=jax.ShapeDtypeStruct((t, _D_MODEL), jnp.float32),
        scratch_shapes=[pltpu.VMEM((_EB, _D_EMB), jnp.float32)],
    )(x, W_hash, basis2, W_dec)


# speedup vs baseline: 3.2239x; 1.5912x over previous
import jax
import jax.numpy as jnp
from jax.experimental import pallas as pl
from jax.experimental.pallas import tpu as pltpu

_D_MODEL = 768
_D_EMB = 64
_E = 64
_BASIS = 8
_EB = _E * _BASIS
_TM = 2048
_CHUNKS = 4


def _hrn_block(x_ref, wh_ref, basis_ref, wdec_ref, y_ref, flat_ref):
    f32 = jnp.float32

    @pl.when(pl.program_id(0) == 0)
    def _init():
        basis = basis_ref[...]
        flat_ref[...] = basis * (1.0 / (
            jnp.sqrt(jnp.sum(basis * basis, axis=1, keepdims=True)) + 1e-8))

    wh = wh_ref[...]
    wdec = wdec_ref[...]
    flat = flat_ref[...]

    rows = _TM // _CHUNKS
    R = range(_CHUNKS)

    eTs = [jax.lax.dot_general(wh, x_ref[pl.ds(h * rows, rows), :],
                               (((0,), (1,)), ((), ())),
                               preferred_element_type=f32) for h in R]
    eTs = [eT * (1.0 / (jnp.sqrt(jnp.sum(eT * eT, axis=0, keepdims=True)) + 1e-8))
           for eT in eTs]

    cTs = [jax.lax.dot_general(flat, eT, (((1,), (0,)), ((), ())),
                               preferred_element_type=f32) for eT in eTs]

    sqs = [c * c for c in cTs]
    s4s = [s[0:4 * _E, :] + s[4 * _E:8 * _E, :] for s in sqs]
    s2s = [s[0:2 * _E, :] + s[2 * _E:4 * _E, :] for s in s4s]
    ens = [s[0:_E, :] + s[_E:2 * _E, :] for s in s2s]

    ws = []
    for en in ens:
        m1 = jnp.max(en, axis=0, keepdims=True)
        is1 = en == m1
        en2 = jnp.where(is1, -1.0, en)
        m2 = jnp.max(en2, axis=0, keepdims=True)
        ed = jnp.exp(m2 - m1)
        g1 = 1.0 / (1.0 + ed)
        g2 = ed * g1
        ws.append(jnp.where(is1, g1, jnp.where(en2 == m2, g2, 0.0)))

    cwTs = [c * jnp.concatenate([w] * _BASIS, axis=0) for c, w in zip(cTs, ws)]
    mixes = [jax.lax.dot_general(cwT, flat, (((0,), (0,)), ((), ())),
                                 preferred_element_type=f32) for cwT in cwTs]

    for h in R:
        y_ref[pl.ds(h * rows, rows), :] = jax.lax.dot_general(
            mixes[h], wdec, (((1,), (0,)), ((), ())), preferred_element_type=f32)


@jax.jit
def kernel(x, W_hash, basis, W_dec):
    t = x.shape[0]
    basis2 = basis.transpose(1, 0, 2).reshape(_EB, _D_EMB)
    return pl.pallas_call(
        _hrn_block,
        grid=(t // _TM,),
        in_specs=[
            pl.BlockSpec((_TM, _D_MODEL), lambda i: (i, 0)),
            pl.BlockSpec((_D_MODEL, _D_EMB), lambda i: (0, 0)),
            pl.BlockSpec((_EB, _D_EMB), lambda i: (0, 0)),
            pl.BlockSpec((_D_EMB, _D_MODEL), lambda i: (0, 0)),
        ],
        out_specs=pl.BlockSpec((_TM, _D_MODEL), lambda i: (i, 0)),
        out_shape=jax.ShapeDtypeStruct((t, _D_MODEL), jnp.float32),
        scratch_shapes=[pltpu.VMEM((_EB, _D_EMB), jnp.float32)],
    )(x, W_hash, basis2, W_dec)


# speedup vs baseline: 3.2245x; 1.0002x over previous
import jax
import jax.numpy as jnp
from jax.experimental import pallas as pl
from jax.experimental.pallas import tpu as pltpu

_D_MODEL = 768
_D_EMB = 64
_E = 64
_BASIS = 8
_EB = _E * _BASIS
_TM = 2048
_CHUNKS = 2


def _hrn_block(x_ref, wh_ref, basis_ref, wdec_ref, y_ref, flat_ref):
    f32 = jnp.float32

    @pl.when(pl.program_id(0) == 0)
    def _init():
        basis = basis_ref[...]
        flat_ref[...] = basis * (1.0 / (
            jnp.sqrt(jnp.sum(basis * basis, axis=1, keepdims=True)) + 1e-8))

    wh = wh_ref[...]
    wdec = wdec_ref[...]
    flat = flat_ref[...]

    rows = _TM // _CHUNKS
    R = range(_CHUNKS)

    eTs = [jax.lax.dot_general(wh, x_ref[pl.ds(h * rows, rows), :],
                               (((0,), (1,)), ((), ())),
                               preferred_element_type=f32) for h in R]
    eTs = [eT * (1.0 / (jnp.sqrt(jnp.sum(eT * eT, axis=0, keepdims=True)) + 1e-8))
           for eT in eTs]

    cTs = [jax.lax.dot_general(flat, eT, (((1,), (0,)), ((), ())),
                               preferred_element_type=f32) for eT in eTs]

    sqs = [c * c for c in cTs]
    s4s = [s[0:4 * _E, :] + s[4 * _E:8 * _E, :] for s in sqs]
    s2s = [s[0:2 * _E, :] + s[2 * _E:4 * _E, :] for s in s4s]
    ens = [s[0:_E, :] + s[_E:2 * _E, :] for s in s2s]

    ws = []
    for en in ens:
        m1 = jnp.max(en, axis=0, keepdims=True)
        is1 = en == m1
        en2 = jnp.where(is1, -1.0, en)
        m2 = jnp.max(en2, axis=0, keepdims=True)
        ed = jnp.exp(m2 - m1)
        g1 = 1.0 / (1.0 + ed)
        g2 = ed * g1
        ws.append(jnp.where(is1, g1, jnp.where(en2 == m2, g2, 0.0)))

    cwTs = [jnp.concatenate([c[b * _E:(b + 1) * _E, :] * w for b in range(_BASIS)],
                            axis=0) for c, w in zip(cTs, ws)]
    mixes = [jax.lax.dot_general(cwT, flat, (((0,), (0,)), ((), ())),
                                 preferred_element_type=f32) for cwT in cwTs]

    for h in R:
        y_ref[pl.ds(h * rows, rows), :] = jax.lax.dot_general(
            mixes[h], wdec, (((1,), (0,)), ((), ())), preferred_element_type=f32)


@jax.jit
def kernel(x, W_hash, basis, W_dec):
    t = x.shape[0]
    basis2 = basis.transpose(1, 0, 2).reshape(_EB, _D_EMB)
    return pl.pallas_call(
        _hrn_block,
        grid=(t // _TM,),
        in_specs=[
            pl.BlockSpec((_TM, _D_MODEL), lambda i: (i, 0)),
            pl.BlockSpec((_D_MODEL, _D_EMB), lambda i: (0, 0)),
            pl.BlockSpec((_EB, _D_EMB), lambda i: (0, 0)),
            pl.BlockSpec((_D_EMB, _D_MODEL), lambda i: (0, 0)),
        ],
        out_specs=pl.BlockSpec((_TM, _D_MODEL), lambda i: (i, 0)),
        out_shape=jax.ShapeDtypeStruct((t, _D_MODEL), jnp.float32),
        scratch_shapes=[pltpu.VMEM((_EB, _D_EMB), jnp.float32)],
    )(x, W_hash, basis2, W_dec)
